# Initial kernel scaffold; baseline (speedup 1.0000x reference)
import jax
import jax.numpy as jnp
from jax.experimental import pallas as pl


def kernel(feat, proposals, W6, b6, W7, b7, Wc, bc, Wb, bb, image_h, image_w):
    raise NotImplementedError("write your pallas kernel here")


# speedup vs baseline: 6.6064x; 6.6064x over previous
import math

import jax
import jax.numpy as jnp
from jax import lax
from jax.experimental import pallas as pl
from jax.experimental.pallas import tpu as pltpu
from jax.experimental.pallas import tpu_sc as plsc

NUM_CLASSES = 21
C_IN = 256
POOL = 7
FC_DIM = 1024
N_PROP = 1000
FEAT_HW = 50
NMS_THRESH = 0.5
TOPK = 100
SCORE_THRESH = 0.05
LOG_MAX = math.log(1000.0 / 16)

NPAD = 1024
Q = POOL * POOL
QPAD = 64
G = 2 * POOL
KDIM = C_IN * Q
NW = 32
PPW = NPAD // NW
NFLAT = N_PROP * (NUM_CLASSES - 1)
RROWS = 160
NEG = -jnp.inf


def _sc_pool_body(table_hbm, px1_hbm, py1_hbm, px2_hbm, py2_hbm, scale_hbm,
                  out_hbm,
                  x1_v, y1_v, x2_v, y2_v, scale_v, xi_v, yi_v,
                  j0_v, j1_v, j2_v, j3_v, ridx_v,
                  r0_v, r1_v, r2_v, r3_v, outb_v, sem):
    wid = lax.axis_index("s") * 2 + lax.axis_index("c")
    pbase = wid * PPW
    obase = wid * PPW * QPAD

    pltpu.sync_copy(px1_hbm.at[pl.ds(pbase, PPW)], x1_v)
    pltpu.sync_copy(py1_hbm.at[pl.ds(pbase, PPW)], y1_v)
    pltpu.sync_copy(px2_hbm.at[pl.ds(pbase, PPW)], x2_v)
    pltpu.sync_copy(py2_hbm.at[pl.ds(pbase, PPW)], y2_v)
    pltpu.sync_copy(scale_hbm, scale_v)
    scale = scale_v[...][0]

    lane = lax.iota(jnp.int32, 16)

    for half in range(2):
        hs = pl.ds(half * 16, 16)
        x1h = x1_v[hs] * scale
        y1h = y1_v[hs] * scale
        rwh = jnp.maximum(x2_v[hs] * scale - x1h, 1.0)
        rhh = jnp.maximum(y2_v[hs] * scale - y1h, 1.0)
        for g in range(G):
            tg = (g + 0.5) / G
            xi = jnp.clip((x1h + rwh * tg).astype(jnp.int32), 0, FEAT_HW - 1)
            yi = jnp.clip((y1h + rhh * tg).astype(jnp.int32), 0, FEAT_HW - 1)
            xi_v[pl.ds(g * PPW + half * 16, 16)] = xi
            yi_v[pl.ds(g * PPW + half * 16, 16)] = yi * FEAT_HW

    for e in range(Q):
        py, px = e // POOL, e % POOL
        for half in range(2):
            hs = pl.ds(half * 16, 16)
            base = (pbase + half * 16 + lane) * QPAD + e
            ridx_v[e, hs] = base
            for d, jref in ((0, j0_v), (1, j1_v), (2, j2_v), (3, j3_v)):
                dy, dx = d // 2, d % 2
                yrow = yi_v[pl.ds((2 * py + dy) * PPW + half * 16, 16)]
                xrow = xi_v[pl.ds((2 * px + dx) * PPW + half * 16, 16)]
                jref[pl.ds(e * PPW + half * 16, 16)] = yrow + xrow

    def do_cell(e, _):
        es = pl.ds(e * PPW, PPW)
        cp0 = pltpu.async_copy(table_hbm.at[j0_v.at[es]], r0_v, sem)
        cp1 = pltpu.async_copy(table_hbm.at[j1_v.at[es]], r1_v, sem)
        cp2 = pltpu.async_copy(table_hbm.at[j2_v.at[es]], r2_v, sem)
        cp3 = pltpu.async_copy(table_hbm.at[j3_v.at[es]], r3_v, sem)
        cp0.wait(); cp1.wait(); cp2.wait(); cp3.wait()

        def do_row(r, _):
            for c in range(C_IN // 16):
                cs = pl.ds(c * 16, 16)
                m = jnp.maximum(jnp.maximum(r0_v[r, cs], r1_v[r, cs]),
                                jnp.maximum(r2_v[r, cs], r3_v[r, cs]))
                outb_v[r, cs] = m
            return _

        lax.fori_loop(0, PPW, do_row, None)
        pltpu.async_copy(outb_v, out_hbm.at[ridx_v.at[e]], sem).wait()
        return _

    lax.fori_loop(0, Q, do_cell, None)


def _sc_pool(table, px1, py1, px2, py2, scale_arr):
    mesh = plsc.VectorSubcoreMesh(core_axis_name="c", subcore_axis_name="s",
                                  num_cores=2, num_subcores=16)
    f = pl.kernel(
        _sc_pool_body,
        out_type=jax.ShapeDtypeStruct((NPAD * QPAD, C_IN), jnp.float32),
        mesh=mesh,
        scratch_types=[
            pltpu.VMEM((PPW,), jnp.float32),
            pltpu.VMEM((PPW,), jnp.float32),
            pltpu.VMEM((PPW,), jnp.float32),
            pltpu.VMEM((PPW,), jnp.float32),
            pltpu.VMEM((16,), jnp.float32),
            pltpu.VMEM((G * PPW,), jnp.int32),
            pltpu.VMEM((G * PPW,), jnp.int32),
            pltpu.VMEM((Q * PPW,), jnp.int32),
            pltpu.VMEM((Q * PPW,), jnp.int32),
            pltpu.VMEM((Q * PPW,), jnp.int32),
            pltpu.VMEM((Q * PPW,), jnp.int32),
            pltpu.VMEM((Q, PPW), jnp.int32),
            pltpu.VMEM((PPW, C_IN), jnp.float32),
            pltpu.VMEM((PPW, C_IN), jnp.float32),
            pltpu.VMEM((PPW, C_IN), jnp.float32),
            pltpu.VMEM((PPW, C_IN), jnp.float32),
            pltpu.VMEM((PPW, C_IN), jnp.float32),
            pltpu.SemaphoreType.DMA,
        ],
    )
    return f(table, px1, py1, px2, py2, scale_arr)


NBLK = 256
KBLK = KDIM // 7


def _head_body(pooled_ref, w6_ref, b6_ref, w7_ref, b7_ref, wc_ref, bc_ref,
               wb_ref, bb_ref, props_ref, imh_ref, imw_ref,
               h7_ref, bx1_ref, by1_ref, bx2_ref, by2_ref, sv_ref, fs_ref,
               acc_ref):
    i = pl.program_id(0)
    j = pl.program_id(1)

    acc_blk = acc_ref[pl.ds(j * NBLK, NBLK), :]
    part = jnp.dot(pooled_ref[...], w6_ref[...],
                   preferred_element_type=jnp.float32)
    acc_ref[pl.ds(j * NBLK, NBLK), :] = jnp.where(i == 0, part, acc_blk + part)

    @pl.when(i == 6)
    def _finish():
        h6 = jax.nn.relu(acc_ref[pl.ds(j * NBLK, NBLK), :] + b6_ref[...])
        h7 = jax.nn.relu(jnp.dot(h6, w7_ref[...],
                                 preferred_element_type=jnp.float32) + b7_ref[...])
        h7_ref[...] = h7

        logits = jnp.dot(h7, wc_ref[...], preferred_element_type=jnp.float32) \
            + bc_ref[...]
        m = jnp.max(logits, axis=1, keepdims=True)
        e = jnp.exp(logits - m)
        scores = e / jnp.sum(e, axis=1, keepdims=True)

        braw = jnp.dot(h7, wb_ref[...], preferred_element_type=jnp.float32) \
            + bb_ref[...]
        dxv = braw[:, 0:32]
        dyv = braw[:, 32:64]
        dwv = jnp.minimum(braw[:, 64:96], LOG_MAX)
        dhv = jnp.minimum(braw[:, 96:128], LOG_MAX)

        x1p = props_ref[:, 0:1]
        y1p = props_ref[:, 1:2]
        x2p = props_ref[:, 2:3]
        y2p = props_ref[:, 3:4]
        w_ = x2p - x1p
        h_ = y2p - y1p
        cx = x1p + 0.5 * w_
        cy = y1p + 0.5 * h_

        pcx = dxv * w_ + cx
        pcy = dyv * h_ + cy
        pw = jnp.exp(dwv) * w_
        ph = jnp.exp(dhv) * h_

        imw = imw_ref[0, 0]
        imh = imh_ref[0, 0]
        bx1 = jnp.clip(pcx - 0.5 * pw, 0.0, imw)
        by1 = jnp.clip(pcy - 0.5 * ph, 0.0, imh)
        bx2 = jnp.clip(pcx + 0.5 * pw, 0.0, imw)
        by2 = jnp.clip(pcy + 0.5 * ph, 0.0, imh)
        bx1_ref[...] = bx1
        by1_ref[...] = by1
        bx2_ref[...] = bx2
        by2_ref[...] = by2

        fsv = jnp.concatenate(
            [scores[:, 1:NUM_CLASSES], jnp.zeros((NBLK, 12), jnp.float32)],
            axis=1)
        fs_ref[...] = fsv
        ws = bx2 - bx1
        hs = by2 - by1
        active = (fsv > SCORE_THRESH) & (ws > 1e-2) & (hs > 1e-2)
        sv_ref[...] = jnp.where(active, fsv, NEG)


def _head(pooled_hbm, w6p, b6r, w7t, b7r, wct, bcr, wbt, bbr, props128,
          imh_a, imw_a):
    pooled2d = pooled_hbm.reshape(NPAD, QPAD * C_IN)
    grid = (7, NPAD // NBLK)
    outs = [
        jax.ShapeDtypeStruct((NPAD, FC_DIM), jnp.float32),
        jax.ShapeDtypeStruct((NPAD, 32), jnp.float32),
        jax.ShapeDtypeStruct((NPAD, 32), jnp.float32),
        jax.ShapeDtypeStruct((NPAD, 32), jnp.float32),
        jax.ShapeDtypeStruct((NPAD, 32), jnp.float32),
        jax.ShapeDtypeStruct((NPAD, 32), jnp.float32),
        jax.ShapeDtypeStruct((NPAD, 32), jnp.float32),
    ]
    in_specs = [
        pl.BlockSpec((NBLK, KBLK), lambda i, j: (j, i)),
        pl.BlockSpec((KBLK, FC_DIM), lambda i, j: (i, 0)),
        pl.BlockSpec((1, FC_DIM), lambda i, j: (0, 0)),
        pl.BlockSpec((FC_DIM, FC_DIM), lambda i, j: (0, 0)),
        pl.BlockSpec((1, FC_DIM), lambda i, j: (0, 0)),
        pl.BlockSpec((FC_DIM, 128), lambda i, j: (0, 0)),
        pl.BlockSpec((1, 128), lambda i, j: (0, 0)),
        pl.BlockSpec((FC_DIM, 128), lambda i, j: (0, 0)),
        pl.BlockSpec((1, 128), lambda i, j: (0, 0)),
        pl.BlockSpec((NBLK, 128), lambda i, j: (j, 0)),
        pl.BlockSpec(memory_space=pltpu.SMEM),
        pl.BlockSpec(memory_space=pltpu.SMEM),
    ]
    out_specs = [
        pl.BlockSpec((NBLK, FC_DIM), lambda i, j: (j, 0)),
        pl.BlockSpec((NBLK, 32), lambda i, j: (j, 0)),
        pl.BlockSpec((NBLK, 32), lambda i, j: (j, 0)),
        pl.BlockSpec((NBLK, 32), lambda i, j: (j, 0)),
        pl.BlockSpec((NBLK, 32), lambda i, j: (j, 0)),
        pl.BlockSpec((NBLK, 32), lambda i, j: (j, 0)),
        pl.BlockSpec((NBLK, 32), lambda i, j: (j, 0)),
    ]
    return pl.pallas_call(
        _head_body,
        grid=grid,
        in_specs=in_specs,
        out_specs=out_specs,
        out_shape=outs,
        scratch_shapes=[pltpu.VMEM((NPAD, FC_DIM), jnp.float32)],
    )(pooled2d, w6p, b6r, w7t, b7r, wct, bcr, wbt, bbr, props128, imh_a, imw_a)


def _nms_body(s_ref, x1_ref, y1_ref, x2_ref, y2_ref, fs_ref, h7_ref,
              imh_ref, imw_ref, misc_ref, h7o_ref,
              sv_ref, nx1_ref, ny1_ref, nx2_ref, ny2_ref, ar_ref):
    fi = (lax.broadcasted_iota(jnp.int32, (RROWS, 128), 0) * 128
          + lax.broadcasted_iota(jnp.int32, (RROWS, 128), 1))
    off_unit = jnp.maximum(imh_ref[0, 0], imw_ref[0, 0]) + 2.0
    cls_f = ((fi % 20) + 1).astype(jnp.float32)
    off = cls_f * off_unit

    nx1 = x1_ref[...] + off
    ny1 = y1_ref[...] + off
    nx2 = x2_ref[...] + off
    ny2 = y2_ref[...] + off
    nx1_ref[...] = nx1
    ny1_ref[...] = ny1
    nx2_ref[...] = nx2
    ny2_ref[...] = ny2
    ar_ref[...] = (nx2 - nx1) * (ny2 - ny1)
    sv_ref[...] = s_ref[...]

    lanes = lax.broadcasted_iota(jnp.int32, (1, 128), 1)

    def step(k, _):
        s = sv_ref[...]
        m = jnp.max(s)
        eq = s == m
        ik = jnp.min(jnp.where(eq, fi, jnp.int32(2 ** 30)))
        sel = fi == ik

        def pick(ref):
            return jnp.sum(jnp.where(sel, ref[...], 0.0))

        bx1 = pick(nx1_ref)
        by1 = pick(ny1_ref)
        bx2 = pick(nx2_ref)
        by2 = pick(ny2_ref)
        ab = pick(ar_ref)

        xl = jnp.maximum(bx1, nx1_ref[...])
        yt = jnp.maximum(by1, ny1_ref[...])
        xr = jnp.minimum(bx2, nx2_ref[...])
        yb = jnp.minimum(by2, ny2_ref[...])
        inter = jnp.maximum(xr - xl, 0.0) * jnp.maximum(yb - yt, 0.0)
        iou = inter / (ab + ar_ref[...] - inter)
        ns = jnp.where(iou > NMS_THRESH, NEG, s)
        sv_ref[...] = jnp.where(sel, NEG, ns)

        rx1 = pick(x1_ref)
        ry1 = pick(y1_ref)
        rx2 = pick(x2_ref)
        ry2 = pick(y2_ref)
        rfs = pick(fs_ref)
        rfl = ((ik % 20) + 1).astype(jnp.float32)
        row = (rx1 * (lanes == 0) + ry1 * (lanes == 1) + rx2 * (lanes == 2)
               + ry2 * (lanes == 3) + rfs * (lanes == 4) + rfl * (lanes == 5))
        misc_ref[pl.ds(k, 1), :] = row

        roi = ik // 20
        h7o_ref[pl.ds(k, 1), :] = h7_ref[pl.ds(roi, 1), :]
        return _

    lax.fori_loop(0, TOPK, step, None)


def _nms(s, x1f, y1f, x2f, y2f, fsf, h7f, imh_a, imw_a):
    outs = [
        jax.ShapeDtypeStruct((104, 128), jnp.float32),
        jax.ShapeDtypeStruct((104, FC_DIM), jnp.float32),
    ]
    in_specs = ([pl.BlockSpec((RROWS, 128), lambda: (0, 0))] * 6
                + [pl.BlockSpec((NPAD, FC_DIM), lambda: (0, 0)),
                   pl.BlockSpec(memory_space=pltpu.SMEM),
                   pl.BlockSpec(memory_space=pltpu.SMEM)])
    out_specs = [
        pl.BlockSpec((104, 128), lambda: (0, 0)),
        pl.BlockSpec((104, FC_DIM), lambda: (0, 0)),
    ]
    return pl.pallas_call(
        _nms_body,
        grid=(),
        in_specs=in_specs,
        out_specs=out_specs,
        out_shape=outs,
        scratch_shapes=[pltpu.VMEM((RROWS, 128), jnp.float32)
                        for _ in range(6)],
    )(s, x1f, y1f, x2f, y2f, fsf, h7f, imh_a, imw_a)


def kernel(feat, proposals, W6, b6, W7, b7, Wc, bc, Wb, bb, image_h, image_w):
    f32 = jnp.float32
    imh_f = jnp.asarray(image_h).astype(f32)
    imw_f = jnp.asarray(image_w).astype(f32)
    scale = FEAT_HW / imh_f

    table = jnp.transpose(feat[0].reshape(C_IN, FEAT_HW * FEAT_HW))
    props_pad = jnp.concatenate(
        [proposals, jnp.zeros((NPAD - N_PROP, 4), f32)], axis=0)
    scale_arr = jnp.full((16,), scale, f32)

    pooled = _sc_pool(table, props_pad[:, 0], props_pad[:, 1],
                      props_pad[:, 2], props_pad[:, 3], scale_arr)

    w6p = W6.reshape(FC_DIM, C_IN, POOL, POOL).transpose(2, 3, 1, 0) \
        .reshape(KDIM, FC_DIM)
    b6r = b6.reshape(1, FC_DIM)
    w7t = W7.T
    b7r = b7.reshape(1, FC_DIM)
    wct = jnp.zeros((FC_DIM, 128), f32).at[:, :NUM_CLASSES].set(Wc.T)
    bcr = jnp.full((1, 128), -1e30, f32).at[0, :NUM_CLASSES].set(bc)
    wb3 = Wb.reshape(NUM_CLASSES, 4, FC_DIM)
    bb2 = bb.reshape(NUM_CLASSES, 4)
    wbt = jnp.zeros((FC_DIM, 128), f32)
    bbr = jnp.zeros((1, 128), f32)
    for ci in range(4):
        wbt = wbt.at[:, 32 * ci:32 * ci + 20].set(wb3[1:, ci, :].T)
        bbr = bbr.at[0, 32 * ci:32 * ci + 20].set(bb2[1:, ci])
    props128 = jnp.zeros((NPAD, 128), f32).at[:, :4].set(props_pad)
    imh_a = imh_f.reshape(1, 1)
    imw_a = imw_f.reshape(1, 1)

    h7f, bx1, by1, bx2, by2, sv, fsv = _head(
        pooled, w6p, b6r, w7t, b7r, wct, bcr, wbt, bbr, props128, imh_a, imw_a)

    def flat(a, fill):
        v = a[:N_PROP, :20].reshape(NFLAT)
        return jnp.concatenate(
            [v, jnp.full((RROWS * 128 - NFLAT,), fill, f32)]).reshape(RROWS, 128)

    s = flat(sv, NEG)
    x1f = flat(bx1, 0.0)
    y1f = flat(by1, 0.0)
    x2f = flat(bx2, 0.0)
    y2f = flat(by2, 0.0)
    fsf = flat(fsv, 0.0)

    misc, h7o = _nms(s, x1f, y1f, x2f, y2f, fsf, h7f, imh_a, imw_a)

    fb_out = misc[:TOPK, 0:4]
    fs_out = misc[:TOPK, 4]
    fl_out = misc[:TOPK, 5].astype(jnp.int32)
    h7_out = h7o[:TOPK]
    return fb_out, fs_out, fl_out, h7_out


# speedup vs baseline: 7.6118x; 1.1522x over previous
import math

import jax
import jax.numpy as jnp
from jax import lax
from jax.experimental import pallas as pl
from jax.experimental.pallas import tpu as pltpu
from jax.experimental.pallas import tpu_sc as plsc

NUM_CLASSES = 21
C_IN = 256
POOL = 7
FC_DIM = 1024
N_PROP = 1000
FEAT_HW = 50
NMS_THRESH = 0.5
TOPK = 100
SCORE_THRESH = 0.05
LOG_MAX = math.log(1000.0 / 16)

NPAD = 1024
Q = POOL * POOL
QPAD = 64
G = 2 * POOL
KDIM = C_IN * Q
NW = 32
PPW = NPAD // NW
NFLAT = N_PROP * (NUM_CLASSES - 1)
RROWS = 160
NEG = -jnp.inf


def _sc_pool_body(table_hbm, px1_hbm, py1_hbm, px2_hbm, py2_hbm, scale_hbm,
                  out_hbm,
                  x1_v, y1_v, x2_v, y2_v, scale_v, xi_v, yi_v,
                  j0_v, j1_v, j2_v, j3_v, ridx_v,
                  ra0_v, ra1_v, ra2_v, ra3_v, rb0_v, rb1_v, rb2_v, rb3_v,
                  outa_v, outb_v, semga, semgb, semsa, semsb):
    wid = lax.axis_index("s") * 2 + lax.axis_index("c")
    pbase = wid * PPW

    pltpu.sync_copy(px1_hbm.at[pl.ds(pbase, PPW)], x1_v)
    pltpu.sync_copy(py1_hbm.at[pl.ds(pbase, PPW)], y1_v)
    pltpu.sync_copy(px2_hbm.at[pl.ds(pbase, PPW)], x2_v)
    pltpu.sync_copy(py2_hbm.at[pl.ds(pbase, PPW)], y2_v)
    pltpu.sync_copy(scale_hbm, scale_v)
    scale = scale_v[...][0]

    lane = lax.iota(jnp.int32, 16)

    for half in range(2):
        hs = pl.ds(half * 16, 16)
        x1h = x1_v[hs] * scale
        y1h = y1_v[hs] * scale
        rwh = jnp.maximum(x2_v[hs] * scale - x1h, 1.0)
        rhh = jnp.maximum(y2_v[hs] * scale - y1h, 1.0)
        for g in range(G):
            tg = (g + 0.5) / G
            xi = jnp.clip((x1h + rwh * tg).astype(jnp.int32), 0, FEAT_HW - 1)
            yi = jnp.clip((y1h + rhh * tg).astype(jnp.int32), 0, FEAT_HW - 1)
            xi_v[pl.ds(g * PPW + half * 16, 16)] = xi
            yi_v[pl.ds(g * PPW + half * 16, 16)] = yi * FEAT_HW

    for e in range(Q):
        py, px = e // POOL, e % POOL
        for half in range(2):
            hs = pl.ds(half * 16, 16)
            base = (pbase + half * 16 + lane) * QPAD + e
            ridx_v[e, hs] = base
            for d, jref in ((0, j0_v), (1, j1_v), (2, j2_v), (3, j3_v)):
                dy, dx = d // 2, d % 2
                yrow = yi_v[pl.ds((2 * py + dy) * PPW + half * 16, 16)]
                xrow = xi_v[pl.ds((2 * px + dx) * PPW + half * 16, 16)]
                jref[pl.ds(e * PPW + half * 16, 16)] = yrow + xrow

    jrefs = (j0_v, j1_v, j2_v, j3_v)

    def fire4(bufs, e, semg):
        es = pl.ds(e * PPW, PPW)
        for jref, buf in zip(jrefs, bufs):
            pltpu.async_copy(table_hbm.at[jref.at[es]], buf, semg)

    def wait4(bufs, e, semg):
        es = pl.ds(e * PPW, PPW)
        for jref, buf in zip(jrefs, bufs):
            pltpu.make_async_copy(table_hbm.at[jref.at[es]], buf, semg).wait()

    def compute(bufs, outb):
        def rowgrp(rr, _):
            for u in range(4):
                r = rr * 4 + u
                for c in range(C_IN // 16):
                    cs = pl.ds(c * 16, 16)
                    m = jnp.maximum(
                        jnp.maximum(bufs[0][r, cs], bufs[1][r, cs]),
                        jnp.maximum(bufs[2][r, cs], bufs[3][r, cs]))
                    outb[r, cs] = m
            return _

        lax.fori_loop(0, PPW // 4, rowgrp, None)

    bufs_a = (ra0_v, ra1_v, ra2_v, ra3_v)
    bufs_b = (rb0_v, rb1_v, rb2_v, rb3_v)

    fire4(bufs_a, 0, semga)

    def do_pair(pp, _):
        e0 = 2 * pp
        e1 = 2 * pp + 1

        @pl.when(e1 < Q)
        def _fb():
            fire4(bufs_b, e1, semgb)

        wait4(bufs_a, e0, semga)

        @pl.when(pp > 0)
        def _wsa():
            pltpu.make_async_copy(outa_v, out_hbm.at[ridx_v.at[e0]],
                                  semsa).wait()

        compute(bufs_a, outa_v)
        pltpu.async_copy(outa_v, out_hbm.at[ridx_v.at[e0]], semsa)

        @pl.when(e0 + 2 < Q)
        def _fa():
            fire4(bufs_a, e0 + 2, semga)

        @pl.when(e1 < Q)
        def _pb():
            wait4(bufs_b, e1, semgb)

            @pl.when(pp > 0)
            def _wsb():
                pltpu.make_async_copy(outb_v, out_hbm.at[ridx_v.at[e1]],
                                      semsb).wait()

            compute(bufs_b, outb_v)
            pltpu.async_copy(outb_v, out_hbm.at[ridx_v.at[e1]], semsb)

        return _

    lax.fori_loop(0, (Q + 1) // 2, do_pair, None)
    pltpu.make_async_copy(outa_v, out_hbm.at[ridx_v.at[Q - 1]], semsa).wait()
    pltpu.make_async_copy(outb_v, out_hbm.at[ridx_v.at[Q - 2]], semsb).wait()


def _sc_pool(table, px1, py1, px2, py2, scale_arr):
    mesh = plsc.VectorSubcoreMesh(core_axis_name="c", subcore_axis_name="s",
                                  num_cores=2, num_subcores=16)
    f = pl.kernel(
        _sc_pool_body,
        out_type=jax.ShapeDtypeStruct((NPAD * QPAD, C_IN), jnp.float32),
        mesh=mesh,
        scratch_types=[
            pltpu.VMEM((PPW,), jnp.float32),
            pltpu.VMEM((PPW,), jnp.float32),
            pltpu.VMEM((PPW,), jnp.float32),
            pltpu.VMEM((PPW,), jnp.float32),
            pltpu.VMEM((16,), jnp.float32),
            pltpu.VMEM((G * PPW,), jnp.int32),
            pltpu.VMEM((G * PPW,), jnp.int32),
            pltpu.VMEM((Q * PPW,), jnp.int32),
            pltpu.VMEM((Q * PPW,), jnp.int32),
            pltpu.VMEM((Q * PPW,), jnp.int32),
            pltpu.VMEM((Q * PPW,), jnp.int32),
            pltpu.VMEM((Q, PPW), jnp.int32),
            pltpu.VMEM((PPW, C_IN), jnp.float32),
            pltpu.VMEM((PPW, C_IN), jnp.float32),
            pltpu.VMEM((PPW, C_IN), jnp.float32),
            pltpu.VMEM((PPW, C_IN), jnp.float32),
            pltpu.VMEM((PPW, C_IN), jnp.float32),
            pltpu.VMEM((PPW, C_IN), jnp.float32),
            pltpu.VMEM((PPW, C_IN), jnp.float32),
            pltpu.VMEM((PPW, C_IN), jnp.float32),
            pltpu.VMEM((PPW, C_IN), jnp.float32),
            pltpu.VMEM((PPW, C_IN), jnp.float32),
            pltpu.SemaphoreType.DMA,
            pltpu.SemaphoreType.DMA,
            pltpu.SemaphoreType.DMA,
            pltpu.SemaphoreType.DMA,
        ],
    )
    return f(table, px1, py1, px2, py2, scale_arr)


NBLK = 256
KBLK = KDIM // 7


def _head_body(pooled_ref, w6_ref, b6_ref, w7_ref, b7_ref, wc_ref, bc_ref,
               wb_ref, bb_ref, props_ref, imh_ref, imw_ref,
               h7_ref, bx1_ref, by1_ref, bx2_ref, by2_ref, sv_ref, fs_ref,
               acc_ref):
    i = pl.program_id(0)
    j = pl.program_id(1)

    acc_blk = acc_ref[pl.ds(j * NBLK, NBLK), :]
    part = jnp.dot(pooled_ref[...], w6_ref[...],
                   preferred_element_type=jnp.float32)
    acc_ref[pl.ds(j * NBLK, NBLK), :] = jnp.where(i == 0, part, acc_blk + part)

    @pl.when(i == 6)
    def _finish():
        h6 = jax.nn.relu(acc_ref[pl.ds(j * NBLK, NBLK), :] + b6_ref[...])
        h7 = jax.nn.relu(jnp.dot(h6, w7_ref[...],
                                 preferred_element_type=jnp.float32) + b7_ref[...])
        h7_ref[...] = h7

        logits = jnp.dot(h7, wc_ref[...], preferred_element_type=jnp.float32) \
            + bc_ref[...]
        m = jnp.max(logits, axis=1, keepdims=True)
        e = jnp.exp(logits - m)
        scores = e / jnp.sum(e, axis=1, keepdims=True)

        braw = jnp.dot(h7, wb_ref[...], preferred_element_type=jnp.float32) \
            + bb_ref[...]
        dxv = braw[:, 0:32]
        dyv = braw[:, 32:64]
        dwv = jnp.minimum(braw[:, 64:96], LOG_MAX)
        dhv = jnp.minimum(braw[:, 96:128], LOG_MAX)

        x1p = props_ref[:, 0:1]
        y1p = props_ref[:, 1:2]
        x2p = props_ref[:, 2:3]
        y2p = props_ref[:, 3:4]
        w_ = x2p - x1p
        h_ = y2p - y1p
        cx = x1p + 0.5 * w_
        cy = y1p + 0.5 * h_

        pcx = dxv * w_ + cx
        pcy = dyv * h_ + cy
        pw = jnp.exp(dwv) * w_
        ph = jnp.exp(dhv) * h_

        imw = imw_ref[0, 0]
        imh = imh_ref[0, 0]
        bx1 = jnp.clip(pcx - 0.5 * pw, 0.0, imw)
        by1 = jnp.clip(pcy - 0.5 * ph, 0.0, imh)
        bx2 = jnp.clip(pcx + 0.5 * pw, 0.0, imw)
        by2 = jnp.clip(pcy + 0.5 * ph, 0.0, imh)
        bx1_ref[...] = bx1
        by1_ref[...] = by1
        bx2_ref[...] = bx2
        by2_ref[...] = by2

        fsv = jnp.concatenate(
            [scores[:, 1:NUM_CLASSES], jnp.zeros((NBLK, 12), jnp.float32)],
            axis=1)
        fs_ref[...] = fsv
        ws = bx2 - bx1
        hs = by2 - by1
        active = (fsv > SCORE_THRESH) & (ws > 1e-2) & (hs > 1e-2)
        sv_ref[...] = jnp.where(active, fsv, NEG)


def _head(pooled_hbm, w6p, b6r, w7t, b7r, wct, bcr, wbt, bbr, props128,
          imh_a, imw_a):
    pooled2d = pooled_hbm.reshape(NPAD, QPAD * C_IN)
    grid = (7, NPAD // NBLK)
    outs = [
        jax.ShapeDtypeStruct((NPAD, FC_DIM), jnp.float32),
        jax.ShapeDtypeStruct((NPAD, 32), jnp.float32),
        jax.ShapeDtypeStruct((NPAD, 32), jnp.float32),
        jax.ShapeDtypeStruct((NPAD, 32), jnp.float32),
        jax.ShapeDtypeStruct((NPAD, 32), jnp.float32),
        jax.ShapeDtypeStruct((NPAD, 32), jnp.float32),
        jax.ShapeDtypeStruct((NPAD, 32), jnp.float32),
    ]
    in_specs = [
        pl.BlockSpec((NBLK, KBLK), lambda i, j: (j, i)),
        pl.BlockSpec((KBLK, FC_DIM), lambda i, j: (i, 0)),
        pl.BlockSpec((1, FC_DIM), lambda i, j: (0, 0)),
        pl.BlockSpec((FC_DIM, FC_DIM), lambda i, j: (0, 0)),
        pl.BlockSpec((1, FC_DIM), lambda i, j: (0, 0)),
        pl.BlockSpec((FC_DIM, 128), lambda i, j: (0, 0)),
        pl.BlockSpec((1, 128), lambda i, j: (0, 0)),
        pl.BlockSpec((FC_DIM, 128), lambda i, j: (0, 0)),
        pl.BlockSpec((1, 128), lambda i, j: (0, 0)),
        pl.BlockSpec((NBLK, 128), lambda i, j: (j, 0)),
        pl.BlockSpec(memory_space=pltpu.SMEM),
        pl.BlockSpec(memory_space=pltpu.SMEM),
    ]
    out_specs = [
        pl.BlockSpec((NBLK, FC_DIM), lambda i, j: (j, 0)),
        pl.BlockSpec((NBLK, 32), lambda i, j: (j, 0)),
        pl.BlockSpec((NBLK, 32), lambda i, j: (j, 0)),
        pl.BlockSpec((NBLK, 32), lambda i, j: (j, 0)),
        pl.BlockSpec((NBLK, 32), lambda i, j: (j, 0)),
        pl.BlockSpec((NBLK, 32), lambda i, j: (j, 0)),
        pl.BlockSpec((NBLK, 32), lambda i, j: (j, 0)),
    ]
    return pl.pallas_call(
        _head_body,
        grid=grid,
        in_specs=in_specs,
        out_specs=out_specs,
        out_shape=outs,
        scratch_shapes=[pltpu.VMEM((NPAD, FC_DIM), jnp.float32)],
    )(pooled2d, w6p, b6r, w7t, b7r, wct, bcr, wbt, bbr, props128, imh_a, imw_a)


def _nms_body(s_ref, x1_ref, y1_ref, x2_ref, y2_ref, fs_ref, h7_ref,
              imh_ref, imw_ref, misc_ref, h7o_ref,
              sv_ref, nx1_ref, ny1_ref, nx2_ref, ny2_ref, ar_ref):
    fi = (lax.broadcasted_iota(jnp.int32, (RROWS, 128), 0) * 128
          + lax.broadcasted_iota(jnp.int32, (RROWS, 128), 1))
    off_unit = jnp.maximum(imh_ref[0, 0], imw_ref[0, 0]) + 2.0
    cls_f = ((fi % 20) + 1).astype(jnp.float32)
    off = cls_f * off_unit

    nx1 = x1_ref[...] + off
    ny1 = y1_ref[...] + off
    nx2 = x2_ref[...] + off
    ny2 = y2_ref[...] + off
    nx1_ref[...] = nx1
    ny1_ref[...] = ny1
    nx2_ref[...] = nx2
    ny2_ref[...] = ny2
    ar_ref[...] = (nx2 - nx1) * (ny2 - ny1)
    sv_ref[...] = s_ref[...]

    lanes = lax.broadcasted_iota(jnp.int32, (1, 128), 1)

    def step(k, _):
        s = sv_ref[...]
        m = jnp.max(s)
        eq = s == m
        ik = jnp.min(jnp.where(eq, fi, jnp.int32(2 ** 30)))
        sel = fi == ik

        def pick(ref):
            return jnp.sum(jnp.where(sel, ref[...], 0.0))

        bx1 = pick(nx1_ref)
        by1 = pick(ny1_ref)
        bx2 = pick(nx2_ref)
        by2 = pick(ny2_ref)
        ab = pick(ar_ref)

        xl = jnp.maximum(bx1, nx1_ref[...])
        yt = jnp.maximum(by1, ny1_ref[...])
        xr = jnp.minimum(bx2, nx2_ref[...])
        yb = jnp.minimum(by2, ny2_ref[...])
        inter = jnp.maximum(xr - xl, 0.0) * jnp.maximum(yb - yt, 0.0)
        iou = inter / (ab + ar_ref[...] - inter)
        ns = jnp.where(iou > NMS_THRESH, NEG, s)
        sv_ref[...] = jnp.where(sel, NEG, ns)

        rx1 = pick(x1_ref)
        ry1 = pick(y1_ref)
        rx2 = pick(x2_ref)
        ry2 = pick(y2_ref)
        rfs = pick(fs_ref)
        rfl = ((ik % 20) + 1).astype(jnp.float32)
        row = (rx1 * (lanes == 0) + ry1 * (lanes == 1) + rx2 * (lanes == 2)
               + ry2 * (lanes == 3) + rfs * (lanes == 4) + rfl * (lanes == 5))
        misc_ref[pl.ds(k, 1), :] = row

        roi = ik // 20
        h7o_ref[pl.ds(k, 1), :] = h7_ref[pl.ds(roi, 1), :]
        return _

    lax.fori_loop(0, TOPK, step, None)


def _nms(s, x1f, y1f, x2f, y2f, fsf, h7f, imh_a, imw_a):
    outs = [
        jax.ShapeDtypeStruct((104, 128), jnp.float32),
        jax.ShapeDtypeStruct((104, FC_DIM), jnp.float32),
    ]
    in_specs = ([pl.BlockSpec((RROWS, 128), lambda: (0, 0))] * 6
                + [pl.BlockSpec((NPAD, FC_DIM), lambda: (0, 0)),
                   pl.BlockSpec(memory_space=pltpu.SMEM),
                   pl.BlockSpec(memory_space=pltpu.SMEM)])
    out_specs = [
        pl.BlockSpec((104, 128), lambda: (0, 0)),
        pl.BlockSpec((104, FC_DIM), lambda: (0, 0)),
    ]
    return pl.pallas_call(
        _nms_body,
        grid=(),
        in_specs=in_specs,
        out_specs=out_specs,
        out_shape=outs,
        scratch_shapes=[pltpu.VMEM((RROWS, 128), jnp.float32)
                        for _ in range(6)],
    )(s, x1f, y1f, x2f, y2f, fsf, h7f, imh_a, imw_a)


def kernel(feat, proposals, W6, b6, W7, b7, Wc, bc, Wb, bb, image_h, image_w):
    f32 = jnp.float32
    imh_f = jnp.asarray(image_h).astype(f32)
    imw_f = jnp.asarray(image_w).astype(f32)
    scale = FEAT_HW / imh_f

    table = jnp.transpose(feat[0].reshape(C_IN, FEAT_HW * FEAT_HW))
    props_pad = jnp.concatenate(
        [proposals, jnp.zeros((NPAD - N_PROP, 4), f32)], axis=0)
    scale_arr = jnp.full((16,), scale, f32)

    pooled = _sc_pool(table, props_pad[:, 0], props_pad[:, 1],
                      props_pad[:, 2], props_pad[:, 3], scale_arr)

    w6p = W6.reshape(FC_DIM, C_IN, POOL, POOL).transpose(2, 3, 1, 0) \
        .reshape(KDIM, FC_DIM)
    b6r = b6.reshape(1, FC_DIM)
    w7t = W7.T
    b7r = b7.reshape(1, FC_DIM)
    wct = jnp.zeros((FC_DIM, 128), f32).at[:, :NUM_CLASSES].set(Wc.T)
    bcr = jnp.full((1, 128), -1e30, f32).at[0, :NUM_CLASSES].set(bc)
    wb3 = Wb.reshape(NUM_CLASSES, 4, FC_DIM)
    bb2 = bb.reshape(NUM_CLASSES, 4)
    wbt = jnp.zeros((FC_DIM, 128), f32)
    bbr = jnp.zeros((1, 128), f32)
    for ci in range(4):
        wbt = wbt.at[:, 32 * ci:32 * ci + 20].set(wb3[1:, ci, :].T)
        bbr = bbr.at[0, 32 * ci:32 * ci + 20].set(bb2[1:, ci])
    props128 = jnp.zeros((NPAD, 128), f32).at[:, :4].set(props_pad)
    imh_a = imh_f.reshape(1, 1)
    imw_a = imw_f.reshape(1, 1)

    h7f, bx1, by1, bx2, by2, sv, fsv = _head(
        pooled, w6p, b6r, w7t, b7r, wct, bcr, wbt, bbr, props128, imh_a, imw_a)

    def flat(a, fill):
        v = a[:N_PROP, :20].reshape(NFLAT)
        return jnp.concatenate(
            [v, jnp.full((RROWS * 128 - NFLAT,), fill, f32)]).reshape(RROWS, 128)

    s = flat(sv, NEG)
    x1f = flat(bx1, 0.0)
    y1f = flat(by1, 0.0)
    x2f = flat(bx2, 0.0)
    y2f = flat(by2, 0.0)
    fsf = flat(fsv, 0.0)

    misc, h7o = _nms(s, x1f, y1f, x2f, y2f, fsf, h7f, imh_a, imw_a)

    fb_out = misc[:TOPK, 0:4]
    fs_out = misc[:TOPK, 4]
    fl_out = misc[:TOPK, 5].astype(jnp.int32)
    h7_out = h7o[:TOPK]
    return fb_out, fs_out, fl_out, h7_out


# speedup vs baseline: 7.6536x; 1.0055x over previous
import math

import jax
import jax.numpy as jnp
from jax import lax
from jax.experimental import pallas as pl
from jax.experimental.pallas import tpu as pltpu
from jax.experimental.pallas import tpu_sc as plsc

NUM_CLASSES = 21
C_IN = 256
POOL = 7
FC_DIM = 1024
N_PROP = 1000
FEAT_HW = 50
NMS_THRESH = 0.5
TOPK = 100
SCORE_THRESH = 0.05
LOG_MAX = math.log(1000.0 / 16)

NPAD = 1024
Q = POOL * POOL
QPAD = 64
G = 2 * POOL
KDIM = C_IN * Q
NW = 32
PPW = NPAD // NW
NFLAT = N_PROP * (NUM_CLASSES - 1)
RROWS = 160
NEG = -jnp.inf


def _sc_pool_body(table_hbm, px1_hbm, py1_hbm, px2_hbm, py2_hbm, scale_hbm,
                  out_hbm,
                  x1_v, y1_v, x2_v, y2_v, scale_v, xi_v, yi_v,
                  j0_v, j1_v, j2_v, j3_v, ridx_v,
                  ra0_v, ra1_v, ra2_v, ra3_v, rb0_v, rb1_v, rb2_v, rb3_v,
                  outa_v, outb_v, semga, semgb, semsa, semsb):
    wid = lax.axis_index("s") * 2 + lax.axis_index("c")
    pbase = wid * PPW

    pltpu.sync_copy(px1_hbm.at[pl.ds(pbase, PPW)], x1_v)
    pltpu.sync_copy(py1_hbm.at[pl.ds(pbase, PPW)], y1_v)
    pltpu.sync_copy(px2_hbm.at[pl.ds(pbase, PPW)], x2_v)
    pltpu.sync_copy(py2_hbm.at[pl.ds(pbase, PPW)], y2_v)
    pltpu.sync_copy(scale_hbm, scale_v)
    scale = scale_v[...][0]

    lane = lax.iota(jnp.int32, 16)

    for half in range(2):
        hs = pl.ds(half * 16, 16)
        x1h = x1_v[hs] * scale
        y1h = y1_v[hs] * scale
        rwh = jnp.maximum(x2_v[hs] * scale - x1h, 1.0)
        rhh = jnp.maximum(y2_v[hs] * scale - y1h, 1.0)
        for g in range(G):
            tg = (g + 0.5) / G
            xi = jnp.clip((x1h + rwh * tg).astype(jnp.int32), 0, FEAT_HW - 1)
            yi = jnp.clip((y1h + rhh * tg).astype(jnp.int32), 0, FEAT_HW - 1)
            xi_v[pl.ds(g * PPW + half * 16, 16)] = xi
            yi_v[pl.ds(g * PPW + half * 16, 16)] = yi * FEAT_HW

    for e in range(Q):
        py, px = e // POOL, e % POOL
        for half in range(2):
            hs = pl.ds(half * 16, 16)
            base = (pbase + half * 16 + lane) * QPAD + e
            ridx_v[e, hs] = base
            for d, jref in ((0, j0_v), (1, j1_v), (2, j2_v), (3, j3_v)):
                dy, dx = d // 2, d % 2
                yrow = yi_v[pl.ds((2 * py + dy) * PPW + half * 16, 16)]
                xrow = xi_v[pl.ds((2 * px + dx) * PPW + half * 16, 16)]
                jref[pl.ds(e * PPW + half * 16, 16)] = yrow + xrow

    jrefs = (j0_v, j1_v, j2_v, j3_v)

    def fire4(bufs, e, semg):
        es = pl.ds(e * PPW, PPW)
        for jref, buf in zip(jrefs, bufs):
            pltpu.async_copy(table_hbm.at[jref.at[es]], buf, semg)

    def wait4(bufs, e, semg):
        es = pl.ds(e * PPW, PPW)
        for jref, buf in zip(jrefs, bufs):
            pltpu.make_async_copy(table_hbm.at[jref.at[es]], buf, semg).wait()

    def compute(bufs, outb):
        def rowgrp(rr, _):
            for u in range(4):
                r = rr * 4 + u
                for c in range(C_IN // 16):
                    cs = pl.ds(c * 16, 16)
                    m = jnp.maximum(
                        jnp.maximum(bufs[0][r, cs], bufs[1][r, cs]),
                        jnp.maximum(bufs[2][r, cs], bufs[3][r, cs]))
                    outb[r, cs] = m
            return _

        lax.fori_loop(0, PPW // 4, rowgrp, None)

    bufs_a = (ra0_v, ra1_v, ra2_v, ra3_v)
    bufs_b = (rb0_v, rb1_v, rb2_v, rb3_v)

    fire4(bufs_a, 0, semga)

    def do_pair(pp, _):
        e0 = 2 * pp
        e1 = 2 * pp + 1

        @pl.when(e1 < Q)
        def _fb():
            fire4(bufs_b, e1, semgb)

        wait4(bufs_a, e0, semga)

        @pl.when(pp > 0)
        def _wsa():
            pltpu.make_async_copy(outa_v, out_hbm.at[ridx_v.at[e0]],
                                  semsa).wait()

        compute(bufs_a, outa_v)
        pltpu.async_copy(outa_v, out_hbm.at[ridx_v.at[e0]], semsa)

        @pl.when(e0 + 2 < Q)
        def _fa():
            fire4(bufs_a, e0 + 2, semga)

        @pl.when(e1 < Q)
        def _pb():
            wait4(bufs_b, e1, semgb)

            @pl.when(pp > 0)
            def _wsb():
                pltpu.make_async_copy(outb_v, out_hbm.at[ridx_v.at[e1]],
                                      semsb).wait()

            compute(bufs_b, outb_v)
            pltpu.async_copy(outb_v, out_hbm.at[ridx_v.at[e1]], semsb)

        return _

    lax.fori_loop(0, (Q + 1) // 2, do_pair, None)
    pltpu.make_async_copy(outa_v, out_hbm.at[ridx_v.at[Q - 1]], semsa).wait()
    pltpu.make_async_copy(outb_v, out_hbm.at[ridx_v.at[Q - 2]], semsb).wait()


def _sc_pool(table, px1, py1, px2, py2, scale_arr):
    mesh = plsc.VectorSubcoreMesh(core_axis_name="c", subcore_axis_name="s",
                                  num_cores=2, num_subcores=16)
    f = pl.kernel(
        _sc_pool_body,
        out_type=jax.ShapeDtypeStruct((NPAD * QPAD, C_IN), jnp.float32),
        mesh=mesh,
        scratch_types=[
            pltpu.VMEM((PPW,), jnp.float32),
            pltpu.VMEM((PPW,), jnp.float32),
            pltpu.VMEM((PPW,), jnp.float32),
            pltpu.VMEM((PPW,), jnp.float32),
            pltpu.VMEM((16,), jnp.float32),
            pltpu.VMEM((G * PPW,), jnp.int32),
            pltpu.VMEM((G * PPW,), jnp.int32),
            pltpu.VMEM((Q * PPW,), jnp.int32),
            pltpu.VMEM((Q * PPW,), jnp.int32),
            pltpu.VMEM((Q * PPW,), jnp.int32),
            pltpu.VMEM((Q * PPW,), jnp.int32),
            pltpu.VMEM((Q, PPW), jnp.int32),
            pltpu.VMEM((PPW, C_IN), jnp.float32),
            pltpu.VMEM((PPW, C_IN), jnp.float32),
            pltpu.VMEM((PPW, C_IN), jnp.float32),
            pltpu.VMEM((PPW, C_IN), jnp.float32),
            pltpu.VMEM((PPW, C_IN), jnp.float32),
            pltpu.VMEM((PPW, C_IN), jnp.float32),
            pltpu.VMEM((PPW, C_IN), jnp.float32),
            pltpu.VMEM((PPW, C_IN), jnp.float32),
            pltpu.VMEM((PPW, C_IN), jnp.float32),
            pltpu.VMEM((PPW, C_IN), jnp.float32),
            pltpu.SemaphoreType.DMA,
            pltpu.SemaphoreType.DMA,
            pltpu.SemaphoreType.DMA,
            pltpu.SemaphoreType.DMA,
        ],
    )
    return f(table, px1, py1, px2, py2, scale_arr)


NBLK = 256
KBLK = KDIM // 7


def _head_body(pooled_ref, w6_ref, b6_ref, w7_ref, b7_ref, wc_ref, bc_ref,
               wb_ref, bb_ref, props_ref, imh_ref, imw_ref,
               h7_ref, bx1_ref, by1_ref, bx2_ref, by2_ref, sv_ref, fs_ref,
               acc_ref):
    i = pl.program_id(0)
    j = pl.program_id(1)

    acc_blk = acc_ref[pl.ds(j * NBLK, NBLK), :]
    part = jnp.dot(pooled_ref[...], w6_ref[...],
                   preferred_element_type=jnp.float32)
    acc_ref[pl.ds(j * NBLK, NBLK), :] = jnp.where(i == 0, part, acc_blk + part)

    @pl.when(i == 6)
    def _finish():
        h6 = jax.nn.relu(acc_ref[pl.ds(j * NBLK, NBLK), :] + b6_ref[...])
        h7 = jax.nn.relu(jnp.dot(h6, w7_ref[...],
                                 preferred_element_type=jnp.float32) + b7_ref[...])
        h7_ref[...] = h7

        logits = jnp.dot(h7, wc_ref[...], preferred_element_type=jnp.float32) \
            + bc_ref[...]
        m = jnp.max(logits, axis=1, keepdims=True)
        e = jnp.exp(logits - m)
        scores = e / jnp.sum(e, axis=1, keepdims=True)

        braw = jnp.dot(h7, wb_ref[...], preferred_element_type=jnp.float32) \
            + bb_ref[...]
        dxv = braw[:, 0:32]
        dyv = braw[:, 32:64]
        dwv = jnp.minimum(braw[:, 64:96], LOG_MAX)
        dhv = jnp.minimum(braw[:, 96:128], LOG_MAX)

        x1p = props_ref[:, 0:1]
        y1p = props_ref[:, 1:2]
        x2p = props_ref[:, 2:3]
        y2p = props_ref[:, 3:4]
        w_ = x2p - x1p
        h_ = y2p - y1p
        cx = x1p + 0.5 * w_
        cy = y1p + 0.5 * h_

        pcx = dxv * w_ + cx
        pcy = dyv * h_ + cy
        pw = jnp.exp(dwv) * w_
        ph = jnp.exp(dhv) * h_

        imw = imw_ref[0, 0]
        imh = imh_ref[0, 0]
        bx1 = jnp.clip(pcx - 0.5 * pw, 0.0, imw)
        by1 = jnp.clip(pcy - 0.5 * ph, 0.0, imh)
        bx2 = jnp.clip(pcx + 0.5 * pw, 0.0, imw)
        by2 = jnp.clip(pcy + 0.5 * ph, 0.0, imh)
        bx1_ref[...] = bx1
        by1_ref[...] = by1
        bx2_ref[...] = bx2
        by2_ref[...] = by2

        fsv = jnp.concatenate(
            [scores[:, 1:NUM_CLASSES], jnp.zeros((NBLK, 12), jnp.float32)],
            axis=1)
        fs_ref[...] = fsv
        ws = bx2 - bx1
        hs = by2 - by1
        active = (fsv > SCORE_THRESH) & (ws > 1e-2) & (hs > 1e-2)
        sv_ref[...] = jnp.where(active, fsv, NEG)


def _head(pooled_hbm, w6p, b6r, w7t, b7r, wct, bcr, wbt, bbr, props128,
          imh_a, imw_a):
    pooled2d = pooled_hbm.reshape(NPAD, QPAD * C_IN)
    grid = (7, NPAD // NBLK)
    outs = [
        jax.ShapeDtypeStruct((NPAD, FC_DIM), jnp.float32),
        jax.ShapeDtypeStruct((NPAD, 32), jnp.float32),
        jax.ShapeDtypeStruct((NPAD, 32), jnp.float32),
        jax.ShapeDtypeStruct((NPAD, 32), jnp.float32),
        jax.ShapeDtypeStruct((NPAD, 32), jnp.float32),
        jax.ShapeDtypeStruct((NPAD, 32), jnp.float32),
        jax.ShapeDtypeStruct((NPAD, 32), jnp.float32),
    ]
    in_specs = [
        pl.BlockSpec((NBLK, KBLK), lambda i, j: (j, i)),
        pl.BlockSpec((KBLK, FC_DIM), lambda i, j: (i, 0)),
        pl.BlockSpec((1, FC_DIM), lambda i, j: (0, 0)),
        pl.BlockSpec((FC_DIM, FC_DIM), lambda i, j: (0, 0)),
        pl.BlockSpec((1, FC_DIM), lambda i, j: (0, 0)),
        pl.BlockSpec((FC_DIM, 128), lambda i, j: (0, 0)),
        pl.BlockSpec((1, 128), lambda i, j: (0, 0)),
        pl.BlockSpec((FC_DIM, 128), lambda i, j: (0, 0)),
        pl.BlockSpec((1, 128), lambda i, j: (0, 0)),
        pl.BlockSpec((NBLK, 128), lambda i, j: (j, 0)),
        pl.BlockSpec(memory_space=pltpu.SMEM),
        pl.BlockSpec(memory_space=pltpu.SMEM),
    ]
    out_specs = [
        pl.BlockSpec((NBLK, FC_DIM), lambda i, j: (j, 0)),
        pl.BlockSpec((NBLK, 32), lambda i, j: (j, 0)),
        pl.BlockSpec((NBLK, 32), lambda i, j: (j, 0)),
        pl.BlockSpec((NBLK, 32), lambda i, j: (j, 0)),
        pl.BlockSpec((NBLK, 32), lambda i, j: (j, 0)),
        pl.BlockSpec((NBLK, 32), lambda i, j: (j, 0)),
        pl.BlockSpec((NBLK, 32), lambda i, j: (j, 0)),
    ]
    return pl.pallas_call(
        _head_body,
        grid=grid,
        in_specs=in_specs,
        out_specs=out_specs,
        out_shape=outs,
        scratch_shapes=[pltpu.VMEM((NPAD, FC_DIM), jnp.float32)],
    )(pooled2d, w6p, b6r, w7t, b7r, wct, bcr, wbt, bbr, props128, imh_a, imw_a)


def _nms_body(s_ref, x1_ref, y1_ref, x2_ref, y2_ref, fs_ref, h7_ref,
              imh_ref, imw_ref, misc_ref, h7o_ref,
              sv_ref, nx1_ref, ny1_ref, nx2_ref, ny2_ref, ar_ref):
    fi = (lax.broadcasted_iota(jnp.int32, (RROWS, 128), 0) * 128
          + lax.broadcasted_iota(jnp.int32, (RROWS, 128), 1))
    off_unit = jnp.maximum(imh_ref[0, 0], imw_ref[0, 0]) + 2.0
    cls_f = ((fi % 20) + 1).astype(jnp.float32)
    off = cls_f * off_unit

    nx1 = x1_ref[...] + off
    ny1 = y1_ref[...] + off
    nx2 = x2_ref[...] + off
    ny2 = y2_ref[...] + off
    nx1_ref[...] = nx1
    ny1_ref[...] = ny1
    nx2_ref[...] = nx2
    ny2_ref[...] = ny2
    ar_ref[...] = (nx2 - nx1) * (ny2 - ny1)
    sv_ref[...] = s_ref[...]

    lanes = lax.broadcasted_iota(jnp.int32, (1, 128), 1)

    def step(k, _):
        s = sv_ref[...]
        m = jnp.max(s)
        eq = s == m
        ik = jnp.min(jnp.where(eq, fi, jnp.int32(2 ** 30)))
        row = ik // 128
        lsel = lanes == (ik % 128)

        def pick(ref):
            return jnp.sum(jnp.where(lsel, ref[pl.ds(row, 1), :], 0.0))

        bx1 = pick(nx1_ref)
        by1 = pick(ny1_ref)
        bx2 = pick(nx2_ref)
        by2 = pick(ny2_ref)
        ab = pick(ar_ref)

        xl = jnp.maximum(bx1, nx1_ref[...])
        yt = jnp.maximum(by1, ny1_ref[...])
        xr = jnp.minimum(bx2, nx2_ref[...])
        yb = jnp.minimum(by2, ny2_ref[...])
        inter = jnp.maximum(xr - xl, 0.0) * jnp.maximum(yb - yt, 0.0)
        iou = inter / (ab + ar_ref[...] - inter)
        sv_ref[...] = jnp.where((iou > NMS_THRESH) | (fi == ik), NEG, s)

        rx1 = pick(x1_ref)
        ry1 = pick(y1_ref)
        rx2 = pick(x2_ref)
        ry2 = pick(y2_ref)
        rfs = pick(fs_ref)
        rfl = ((ik % 20) + 1).astype(jnp.float32)
        row = (rx1 * (lanes == 0) + ry1 * (lanes == 1) + rx2 * (lanes == 2)
               + ry2 * (lanes == 3) + rfs * (lanes == 4) + rfl * (lanes == 5))
        misc_ref[pl.ds(k, 1), :] = row

        roi = ik // 20
        h7o_ref[pl.ds(k, 1), :] = h7_ref[pl.ds(roi, 1), :]
        return _

    lax.fori_loop(0, TOPK, step, None)


def _nms(s, x1f, y1f, x2f, y2f, fsf, h7f, imh_a, imw_a):
    outs = [
        jax.ShapeDtypeStruct((104, 128), jnp.float32),
        jax.ShapeDtypeStruct((104, FC_DIM), jnp.float32),
    ]
    in_specs = ([pl.BlockSpec((RROWS, 128), lambda: (0, 0))] * 6
                + [pl.BlockSpec((NPAD, FC_DIM), lambda: (0, 0)),
                   pl.BlockSpec(memory_space=pltpu.SMEM),
                   pl.BlockSpec(memory_space=pltpu.SMEM)])
    out_specs = [
        pl.BlockSpec((104, 128), lambda: (0, 0)),
        pl.BlockSpec((104, FC_DIM), lambda: (0, 0)),
    ]
    return pl.pallas_call(
        _nms_body,
        grid=(),
        in_specs=in_specs,
        out_specs=out_specs,
        out_shape=outs,
        scratch_shapes=[pltpu.VMEM((RROWS, 128), jnp.float32)
                        for _ in range(6)],
    )(s, x1f, y1f, x2f, y2f, fsf, h7f, imh_a, imw_a)


def kernel(feat, proposals, W6, b6, W7, b7, Wc, bc, Wb, bb, image_h, image_w):
    f32 = jnp.float32
    imh_f = jnp.asarray(image_h).astype(f32)
    imw_f = jnp.asarray(image_w).astype(f32)
    scale = FEAT_HW / imh_f

    table = jnp.transpose(feat[0].reshape(C_IN, FEAT_HW * FEAT_HW))
    props_pad = jnp.concatenate(
        [proposals, jnp.zeros((NPAD - N_PROP, 4), f32)], axis=0)
    scale_arr = jnp.full((16,), scale, f32)

    pooled = _sc_pool(table, props_pad[:, 0], props_pad[:, 1],
                      props_pad[:, 2], props_pad[:, 3], scale_arr)

    w6p = W6.reshape(FC_DIM, C_IN, POOL, POOL).transpose(2, 3, 1, 0) \
        .reshape(KDIM, FC_DIM)
    b6r = b6.reshape(1, FC_DIM)
    w7t = W7.T
    b7r = b7.reshape(1, FC_DIM)
    wct = jnp.zeros((FC_DIM, 128), f32).at[:, :NUM_CLASSES].set(Wc.T)
    bcr = jnp.full((1, 128), -1e30, f32).at[0, :NUM_CLASSES].set(bc)
    wb3 = Wb.reshape(NUM_CLASSES, 4, FC_DIM)
    bb2 = bb.reshape(NUM_CLASSES, 4)
    wbt = jnp.zeros((FC_DIM, 128), f32)
    bbr = jnp.zeros((1, 128), f32)
    for ci in range(4):
        wbt = wbt.at[:, 32 * ci:32 * ci + 20].set(wb3[1:, ci, :].T)
        bbr = bbr.at[0, 32 * ci:32 * ci + 20].set(bb2[1:, ci])
    props128 = jnp.zeros((NPAD, 128), f32).at[:, :4].set(props_pad)
    imh_a = imh_f.reshape(1, 1)
    imw_a = imw_f.reshape(1, 1)

    h7f, bx1, by1, bx2, by2, sv, fsv = _head(
        pooled, w6p, b6r, w7t, b7r, wct, bcr, wbt, bbr, props128, imh_a, imw_a)

    def flat(a, fill):
        v = a[:N_PROP, :20].reshape(NFLAT)
        return jnp.concatenate(
            [v, jnp.full((RROWS * 128 - NFLAT,), fill, f32)]).reshape(RROWS, 128)

    s = flat(sv, NEG)
    x1f = flat(bx1, 0.0)
    y1f = flat(by1, 0.0)
    x2f = flat(bx2, 0.0)
    y2f = flat(by2, 0.0)
    fsf = flat(fsv, 0.0)

    misc, h7o = _nms(s, x1f, y1f, x2f, y2f, fsf, h7f, imh_a, imw_a)

    fb_out = misc[:TOPK, 0:4]
    fs_out = misc[:TOPK, 4]
    fl_out = misc[:TOPK, 5].astype(jnp.int32)
    h7_out = h7o[:TOPK]
    return fb_out, fs_out, fl_out, h7_out
